# baseline (speedup 1.0000x reference)
import dataclasses
import functools

import jax
import jax.numpy as jnp
from jax.experimental import pallas as pl
from jax.experimental.pallas import tpu as pltpu
from jax.experimental.pallas import tpu_sc as plsc

_P = 8
_TN = 200
_GW = 128
_NW = 32
_VEC = 16
_NCHUNK = 10


def _sc_gather_feats(xt, idx):
    M = idx.shape[1]
    C = xt.shape[1]
    mesh = plsc.VectorSubcoreMesh(core_axis_name="c", subcore_axis_name="s")

    @functools.partial(
        pl.kernel,
        out_type=jax.ShapeDtypeStruct((M, C), xt.dtype),
        mesh=mesh,
    )
    def k(x_hbm, i_hbm, o_hbm):
        def body(i_vmem, o_vmem):
            pltpu.sync_copy(x_hbm.at[i_vmem.at[0]], o_vmem)

        pltpu.emit_pipeline(
            body,
            grid=(M // _GW,),
            in_specs=[pl.BlockSpec((1, _GW), lambda i: (0, i))],
            out_specs=[pl.BlockSpec((_GW, C), lambda i: (i, 0))],
            core_axis_name=("c", "s"),
            dimension_semantics=(pltpu.PARALLEL,),
        )(i_hbm, o_hbm)

    return k(xt, idx)


def _sc_gather_coords(px, py, pz, idx):
    R = px.shape[0]
    M = idx.shape[0]
    per_w = M // _NW
    CH = 2000
    mesh = plsc.VectorSubcoreMesh(core_axis_name="c", subcore_axis_name="s")
    cp = pltpu.CompilerParams()
    if "needs_layout_passes" in pltpu.CompilerParams.__dataclass_fields__:
        cp = dataclasses.replace(cp, needs_layout_passes=False)

    @functools.partial(
        pl.kernel,
        out_type=tuple(jax.ShapeDtypeStruct((M,), jnp.float32)
                       for _ in range(3)),
        mesh=mesh,
        compiler_params=cp,
        scratch_types=[
            pltpu.VMEM((R,), jnp.float32),
            pltpu.VMEM((R,), jnp.float32),
            pltpu.VMEM((R,), jnp.float32),
            pltpu.VMEM((CH,), jnp.int32),
            pltpu.VMEM((CH,), jnp.float32),
            pltpu.VMEM((CH,), jnp.float32),
            pltpu.VMEM((CH,), jnp.float32),
        ],
    )
    def k(px_hbm, py_hbm, pz_hbm, i_hbm, o0_hbm, o1_hbm, o2_hbm,
          tx_v, ty_v, tz_v, i_v, o0_v, o1_v, o2_v):
        wid = jax.lax.axis_index("s") * 2 + jax.lax.axis_index("c")
        base = wid * per_w
        pltpu.sync_copy(px_hbm, tx_v)
        pltpu.sync_copy(py_hbm, ty_v)
        pltpu.sync_copy(pz_hbm, tz_v)

        @pl.loop(0, per_w, step=CH)
        def _chunk(c0):
            pltpu.sync_copy(i_hbm.at[pl.ds(base + c0, CH)], i_v)

            @pl.loop(0, CH, step=_VEC)
            def _vec(t):
                iv = i_v[pl.ds(t, _VEC)]
                o0_v[pl.ds(t, _VEC)] = plsc.load_gather(tx_v, [iv])
                o1_v[pl.ds(t, _VEC)] = plsc.load_gather(ty_v, [iv])
                o2_v[pl.ds(t, _VEC)] = plsc.load_gather(tz_v, [iv])

            pltpu.sync_copy(o0_v, o0_hbm.at[pl.ds(base + c0, CH)])
            pltpu.sync_copy(o1_v, o1_hbm.at[pl.ds(base + c0, CH)])
            pltpu.sync_copy(o2_v, o2_hbm.at[pl.ds(base + c0, CH)])

    return k(px, py, pz, idx)


def _bn_relu(y, g, be):
    cnt = y.shape[0] * y.shape[2]
    s = jnp.sum(y, axis=2, keepdims=True)
    ss = jnp.sum(y * y, axis=2, keepdims=True)
    mean = jnp.sum(s, axis=0, keepdims=True) / cnt
    ex2 = jnp.sum(ss, axis=0, keepdims=True) / cnt
    var = ex2 - mean * mean
    inv = jax.lax.rsqrt(var + 1e-5)
    return jnp.maximum((y - mean) * (inv * g[None]) + be[None], 0.0)


def _mlp_body(m_ref, feat_ref, bn_ref, w1_ref, b1_ref, w2_ref, b2_ref,
              out_ref):
    B, TN, D1 = m_ref.shape
    HID = w1_ref.shape[1]
    OUTD = w2_ref.shape[1]
    C = out_ref.shape[2]
    K = OUTD // C

    dot = functools.partial(jnp.dot, preferred_element_type=jnp.float32)
    y = dot(m_ref[...].reshape(B * TN, D1), w1_ref[...]) + b1_ref[...]
    y = _bn_relu(y.reshape(B, TN, HID), bn_ref[:, 0:1], bn_ref[:, 1:2])
    z = (dot(y.reshape(B * TN, HID).astype(w2_ref.dtype), w2_ref[...])
         + b2_ref[...])
    z = _bn_relu(z.reshape(B, TN, OUTD), bn_ref[:, 2:3], bn_ref[:, 3:4])

    f = feat_ref[...]
    acc = z[:, :, 0:C] * f[:, 0]
    for k in range(1, K):
        acc = acc + z[:, :, k * C:(k + 1) * C] * f[:, k]
    out_ref[...] = acc


def _mlp_call(m, feats, bn, w1, b1, w2t, b2, tile0, ntiles):
    B, _, D1 = m.shape
    K = feats.shape[1]
    C = feats.shape[3]
    HID = w2t.shape[0]
    OUTD = w2t.shape[1]
    return pl.pallas_call(
        _mlp_body,
        grid=(ntiles,),
        in_specs=[
            pl.BlockSpec((B, _TN, D1), lambda i: (0, i + tile0, 0)),
            pl.BlockSpec((B, K, _TN, C), lambda i: (0, 0, i, 0)),
            pl.BlockSpec((_TN, 4), lambda i: (i + tile0, 0)),
            pl.BlockSpec((D1, HID), lambda i: (0, 0)),
            pl.BlockSpec((1, HID), lambda i: (0, 0)),
            pl.BlockSpec((HID, OUTD), lambda i: (0, 0)),
            pl.BlockSpec((1, OUTD), lambda i: (0, 0)),
        ],
        out_specs=pl.BlockSpec((B, _TN, C), lambda i: (0, i, 0)),
        out_shape=jax.ShapeDtypeStruct((B, ntiles * _TN, C), jnp.float32),
    )(m, feats, bn, w1, b1, w2t, b2)


def kernel(x, points, indices, W1, b1, g1, be1, W2, b2, g2, be2):
    B, N, C = x.shape
    K = indices.shape[2]
    HID = W1.shape[0]
    OUTD = W2.shape[0]

    xt = x.reshape(B * N, C)
    pf = points.reshape(B * N, 3)
    idx = (indices.astype(jnp.int32)
           + (jnp.arange(B, dtype=jnp.int32) * N)[:, None, None])
    n0, n1, n2 = _sc_gather_coords(pf[:, 0], pf[:, 1], pf[:, 2],
                                   idx.reshape(B * N * K))

    pts_p = jnp.pad(points, ((0, 0), (0, 0), (0, _P - 3)))
    m = jnp.concatenate(
        [pts_p, n0.reshape(B, N, K), n1.reshape(B, N, K),
         n2.reshape(B, N, K)], axis=2)
    w1_khj = W1.reshape(HID, K, 3)
    w1c = jnp.zeros((_P, HID), W1.dtype).at[:3, :].set(
        jnp.sum(w1_khj, axis=1).T)
    w1 = jnp.concatenate(
        [w1c, -w1_khj[:, :, 0].T, -w1_khj[:, :, 1].T, -w1_khj[:, :, 2].T],
        axis=0)
    w2t = W2.T.astype(jnp.bfloat16)
    bn = jnp.stack([g1, be1, g2, be2], axis=-1)

    nc = N // _NCHUNK
    tiles_per_chunk = nc // _TN
    outs = []
    for c in range(_NCHUNK):
        idx_c = idx[:, c * nc:(c + 1) * nc, :].transpose(0, 2, 1)
        feats_c = _sc_gather_feats(
            xt, idx_c.reshape(1, B * K * nc)).reshape(B, K, nc, C)
        outs.append(_mlp_call(
            m, feats_c, bn, w1, b1.reshape(1, HID), w2t,
            b2.reshape(1, OUTD), c * tiles_per_chunk, tiles_per_chunk))
    out = jnp.concatenate(outs, axis=1)
    return (out, points, indices)


# speedup vs baseline: 1.1183x; 1.1183x over previous
import dataclasses
import functools

import jax
import jax.numpy as jnp
from jax.experimental import pallas as pl
from jax.experimental.pallas import tpu as pltpu
from jax.experimental.pallas import tpu_sc as plsc

_P = 8
_TN = 400
_GW = 128
_NW = 32
_VEC = 16
_CHUNKS = (800, 1200, 2400, 2800, 2800)


def _sc_gather_feats(xt, idx):
    M = idx.shape[1]
    C = xt.shape[1]
    mesh = plsc.VectorSubcoreMesh(core_axis_name="c", subcore_axis_name="s")

    @functools.partial(
        pl.kernel,
        out_type=jax.ShapeDtypeStruct((M, C), xt.dtype),
        mesh=mesh,
    )
    def k(x_hbm, i_hbm, o_hbm):
        def body(i_vmem, o_vmem):
            pltpu.sync_copy(x_hbm.at[i_vmem.at[0]], o_vmem)

        pltpu.emit_pipeline(
            body,
            grid=(M // _GW,),
            in_specs=[pl.BlockSpec((1, _GW), lambda i: (0, i))],
            out_specs=[pl.BlockSpec((_GW, C), lambda i: (i, 0))],
            core_axis_name=("c", "s"),
            dimension_semantics=(pltpu.PARALLEL,),
        )(i_hbm, o_hbm)

    return k(xt, idx)


def _sc_gather_coords(px, py, pz, idx):
    R = px.shape[0]
    M = idx.shape[0]
    per_w = M // _NW
    CH = 2000
    mesh = plsc.VectorSubcoreMesh(core_axis_name="c", subcore_axis_name="s")
    cp = pltpu.CompilerParams()
    if "needs_layout_passes" in pltpu.CompilerParams.__dataclass_fields__:
        cp = dataclasses.replace(cp, needs_layout_passes=False)

    @functools.partial(
        pl.kernel,
        out_type=tuple(jax.ShapeDtypeStruct((M,), jnp.float32)
                       for _ in range(3)),
        mesh=mesh,
        compiler_params=cp,
        scratch_types=[
            pltpu.VMEM((R,), jnp.float32),
            pltpu.VMEM((R,), jnp.float32),
            pltpu.VMEM((R,), jnp.float32),
            pltpu.VMEM((CH,), jnp.int32),
            pltpu.VMEM((CH,), jnp.float32),
            pltpu.VMEM((CH,), jnp.float32),
            pltpu.VMEM((CH,), jnp.float32),
        ],
    )
    def k(px_hbm, py_hbm, pz_hbm, i_hbm, o0_hbm, o1_hbm, o2_hbm,
          tx_v, ty_v, tz_v, i_v, o0_v, o1_v, o2_v):
        wid = jax.lax.axis_index("s") * 2 + jax.lax.axis_index("c")
        base = wid * per_w
        pltpu.sync_copy(px_hbm, tx_v)
        pltpu.sync_copy(py_hbm, ty_v)
        pltpu.sync_copy(pz_hbm, tz_v)

        @pl.loop(0, per_w, step=CH)
        def _chunk(c0):
            pltpu.sync_copy(i_hbm.at[pl.ds(base + c0, CH)], i_v)

            @pl.loop(0, CH, step=_VEC)
            def _vec(t):
                iv = i_v[pl.ds(t, _VEC)]
                o0_v[pl.ds(t, _VEC)] = plsc.load_gather(tx_v, [iv])
                o1_v[pl.ds(t, _VEC)] = plsc.load_gather(ty_v, [iv])
                o2_v[pl.ds(t, _VEC)] = plsc.load_gather(tz_v, [iv])

            pltpu.sync_copy(o0_v, o0_hbm.at[pl.ds(base + c0, CH)])
            pltpu.sync_copy(o1_v, o1_hbm.at[pl.ds(base + c0, CH)])
            pltpu.sync_copy(o2_v, o2_hbm.at[pl.ds(base + c0, CH)])

    return k(px, py, pz, idx)


def _bn_relu(y, g, be):
    cnt = y.shape[0] * y.shape[2]
    s = jnp.sum(y, axis=2, keepdims=True)
    ss = jnp.sum(y * y, axis=2, keepdims=True)
    mean = jnp.sum(s, axis=0, keepdims=True) / cnt
    ex2 = jnp.sum(ss, axis=0, keepdims=True) / cnt
    var = ex2 - mean * mean
    inv = jax.lax.rsqrt(var + 1e-5)
    return jnp.maximum((y - mean) * (inv * g[None]) + be[None], 0.0)


def _mlp_body(m_ref, feat_ref, bn_ref, w1_ref, b1_ref, w2_ref, b2_ref,
              out_ref):
    B, TN, D1 = m_ref.shape
    HID = w1_ref.shape[1]
    OUTD = w2_ref.shape[1]
    C = out_ref.shape[2]
    K = OUTD // C

    dot = functools.partial(jnp.dot, preferred_element_type=jnp.float32)
    y = dot(m_ref[...].reshape(B * TN, D1), w1_ref[...]) + b1_ref[...]
    y = _bn_relu(y.reshape(B, TN, HID), bn_ref[:, 0:1], bn_ref[:, 1:2])
    z = (dot(y.reshape(B * TN, HID).astype(w2_ref.dtype), w2_ref[...])
         + b2_ref[...])
    z = _bn_relu(z.reshape(B, TN, OUTD), bn_ref[:, 2:3], bn_ref[:, 3:4])

    f = feat_ref[...]
    acc = z[:, :, 0:C] * f[:, 0]
    for k in range(1, K):
        acc = acc + z[:, :, k * C:(k + 1) * C] * f[:, k]
    out_ref[...] = acc


def _mlp_call(m, feats, bn, w1, b1, w2t, b2, tile0, ntiles):
    B, _, D1 = m.shape
    K = feats.shape[1]
    C = feats.shape[3]
    HID = w2t.shape[0]
    OUTD = w2t.shape[1]
    return pl.pallas_call(
        _mlp_body,
        grid=(ntiles,),
        in_specs=[
            pl.BlockSpec((B, _TN, D1), lambda i: (0, i + tile0, 0)),
            pl.BlockSpec((B, K, _TN, C), lambda i: (0, 0, i, 0)),
            pl.BlockSpec((_TN, 4), lambda i: (i + tile0, 0)),
            pl.BlockSpec((D1, HID), lambda i: (0, 0)),
            pl.BlockSpec((1, HID), lambda i: (0, 0)),
            pl.BlockSpec((HID, OUTD), lambda i: (0, 0)),
            pl.BlockSpec((1, OUTD), lambda i: (0, 0)),
        ],
        out_specs=pl.BlockSpec((B, _TN, C), lambda i: (0, i, 0)),
        out_shape=jax.ShapeDtypeStruct((B, ntiles * _TN, C), jnp.float32),
    )(m, feats, bn, w1, b1, w2t, b2)


def kernel(x, points, indices, W1, b1, g1, be1, W2, b2, g2, be2):
    B, N, C = x.shape
    K = indices.shape[2]
    HID = W1.shape[0]
    OUTD = W2.shape[0]

    xt = x.reshape(B * N, C)
    pf = points.reshape(B * N, 3)
    idx = (indices.astype(jnp.int32)
           + (jnp.arange(B, dtype=jnp.int32) * N)[:, None, None])
    n0, n1, n2 = _sc_gather_coords(pf[:, 0], pf[:, 1], pf[:, 2],
                                   idx.reshape(B * N * K))

    pts_p = jnp.pad(points, ((0, 0), (0, 0), (0, _P - 3)))
    m = jnp.concatenate(
        [pts_p, n0.reshape(B, N, K), n1.reshape(B, N, K),
         n2.reshape(B, N, K)], axis=2)
    w1_khj = W1.reshape(HID, K, 3)
    w1c = jnp.zeros((_P, HID), W1.dtype).at[:3, :].set(
        jnp.sum(w1_khj, axis=1).T)
    w1 = jnp.concatenate(
        [w1c, -w1_khj[:, :, 0].T, -w1_khj[:, :, 1].T, -w1_khj[:, :, 2].T],
        axis=0)
    w2t = W2.T.astype(jnp.bfloat16)
    bn = jnp.stack([g1, be1, g2, be2], axis=-1)

    chunks = _CHUNKS if sum(_CHUNKS) == N else (N,)
    outs = []
    n0c = 0
    for nc in chunks:
        idx_c = idx[:, n0c:n0c + nc, :].transpose(0, 2, 1)
        feats_c = _sc_gather_feats(
            xt, idx_c.reshape(1, B * K * nc)).reshape(B, K, nc, C)
        outs.append(_mlp_call(
            m, feats_c, bn, w1, b1.reshape(1, HID), w2t,
            b2.reshape(1, OUTD), n0c // _TN, nc // _TN))
        n0c += nc
    out = jnp.concatenate(outs, axis=1)
    return (out, points, indices)
